# baseline (device time: 127868 ns/iter reference)
import jax
import jax.numpy as jnp
from jax import lax
from jax.experimental import pallas as pl
from jax.experimental.pallas import tpu as pltpu

N_DEV = 4


def _neighbor_barrier(left, right):
    barrier_sem = pltpu.get_barrier_semaphore()
    for nbr in (left, right):
        pl.semaphore_signal(
            barrier_sem, inc=1,
            device_id=(nbr,), device_id_type=pl.DeviceIdType.MESH,
        )
    pl.semaphore_wait(barrier_sem, 2)


def _fused(x, W1, W2):
    m_per, n = x.shape
    F = W1.shape[1]
    bf = 1024
    nf = F // bf

    def body(x_ref, w1_ref, w2_ref, out_ref,
             xg, p_buf, sbuf, comm, w1t, w2t,
             ag_send, ag_recv, rs_send, rs_recv, w_sems):
        my = lax.axis_index("i")
        left = (my - 1) % N_DEV
        right = (my + 1) % N_DEV
        _neighbor_barrier(left, right)

        def start_w_load(ft, slot):
            cp1 = pltpu.make_async_copy(
                w1_ref.at[:, pl.ds(ft * bf, bf)], w1t.at[slot],
                w_sems.at[slot, 0],
            )
            cp2 = pltpu.make_async_copy(
                w2_ref.at[pl.ds(ft * bf, bf), :], w2t.at[slot],
                w_sems.at[slot, 1],
            )
            cp1.start()
            cp2.start()
            return cp1, cp2

        pend = {"cps": None}

        def compute_block(b, r0=0, rows=None, pre_next=False, to_out=False):
            rows = m_per if rows is None else rows
            xb = xg[b, pl.ds(r0, rows), :]
            cps = pend["cps"] if pend["cps"] is not None else start_w_load(0, 0)
            pend["cps"] = None
            accv = None
            for ft in range(nf):
                cur = ft % 2
                if ft + 1 < nf:
                    nxt_cps = start_w_load(ft + 1, 1 - cur)
                elif pre_next:
                    pend["cps"] = start_w_load(0, 1 - cur)
                cps[0].wait()
                cps[1].wait()
                h = jnp.dot(
                    xb, w1t[cur], preferred_element_type=jnp.float32
                ).astype(jnp.bfloat16)
                h = h * jax.nn.sigmoid(h)
                c = jnp.dot(
                    h, w2t[cur], preferred_element_type=jnp.float32
                )
                accv = c if accv is None else accv + c
                if ft + 1 < nf:
                    cps = nxt_cps
            if to_out:
                out_ref[pl.ds(r0, rows), :] = accv
            else:
                p_buf[b, pl.ds(r0, rows), :] = accv.astype(jnp.bfloat16)

        def ag_rdma(h, c, target):
            return pltpu.make_async_remote_copy(
                src_ref=xg.at[c], dst_ref=xg.at[c],
                send_sem=ag_send.at[h], recv_sem=ag_recv.at[h],
                device_id=(target,), device_id_type=pl.DeviceIdType.MESH,
            )

        def rs_rdma(s, src, dst, target):
            return pltpu.make_async_remote_copy(
                src_ref=src, dst_ref=dst,
                send_sem=rs_send.at[s], recv_sem=rs_recv.at[s],
                device_id=(target,), device_id_type=pl.DeviceIdType.MESH,
            )

        xg[my] = x_ref[...].astype(jnp.bfloat16)

        a1 = ag_rdma(0, my, left)
        a2 = ag_rdma(1, my, right)
        a1.start()
        a2.start()
        compute_block(my, pre_next=True, to_out=True)

        a1.wait_recv()
        a3 = ag_rdma(2, (my + 1) % N_DEV, left)
        a3.start()
        a2.wait_recv()
        compute_block((my - 1) % N_DEV, pre_next=True)
        r1 = rs_rdma(
            0, p_buf.at[(my - 1) % N_DEV], comm.at[0], left
        )
        r1.start()

        a3.wait_recv()
        compute_block((my + 2) % N_DEV, pre_next=True)
        hm = m_per // 2
        diag = (my + 2) % N_DEV
        last = (my + 1) % N_DEV
        r2a = rs_rdma(
            1, p_buf.at[diag, pl.ds(0, hm)], comm.at[1, pl.ds(0, hm)], right
        )
        r2a.start()
        r2b = rs_rdma(
            2, p_buf.at[diag, pl.ds(hm, hm)], comm.at[1, pl.ds(hm, hm)], right
        )
        r2b.start()

        compute_block(last, 0, hm, pre_next=True)
        r2a.wait_recv()
        sbuf[pl.ds(0, hm), :] = (
            comm[1, pl.ds(0, hm), :].astype(jnp.float32)
            + p_buf[last, pl.ds(0, hm), :].astype(jnp.float32)
        ).astype(jnp.bfloat16)
        r3a = rs_rdma(
            3, sbuf.at[pl.ds(0, hm)], comm.at[2, pl.ds(0, hm)], right
        )
        r3a.start()
        compute_block(last, hm, hm)
        r2b.wait_recv()
        sbuf[pl.ds(hm, hm), :] = (
            comm[1, pl.ds(hm, hm), :].astype(jnp.float32)
            + p_buf[last, pl.ds(hm, hm), :].astype(jnp.float32)
        ).astype(jnp.bfloat16)
        r3b = rs_rdma(
            4, sbuf.at[pl.ds(hm, hm)], comm.at[2, pl.ds(hm, hm)], right
        )
        r3b.start()

        r1.wait_recv()
        r3a.wait_recv()
        r3b.wait_recv()
        out_ref[...] += (
            comm[0].astype(jnp.float32) + comm[2].astype(jnp.float32)
        )
        for d in (a1, a2, a3, r1, r2a, r2b, r3a, r3b):
            d.wait_send()

    return pl.pallas_call(
        body,
        out_shape=jax.ShapeDtypeStruct((m_per, n), jnp.float32),
        in_specs=[
            pl.BlockSpec(memory_space=pltpu.VMEM),
            pl.BlockSpec(memory_space=pl.ANY),
            pl.BlockSpec(memory_space=pl.ANY),
        ],
        out_specs=pl.BlockSpec(memory_space=pltpu.VMEM),
        scratch_shapes=[
            pltpu.VMEM((N_DEV, m_per, n), jnp.bfloat16),
            pltpu.VMEM((N_DEV, m_per, n), jnp.bfloat16),
            pltpu.VMEM((m_per, n), jnp.bfloat16),
            pltpu.VMEM((N_DEV - 1, m_per, n), jnp.bfloat16),
            pltpu.VMEM((2, m_per, bf), jnp.float32),
            pltpu.VMEM((2, bf, n), jnp.float32),
            pltpu.SemaphoreType.DMA((N_DEV - 1,)),
            pltpu.SemaphoreType.DMA((N_DEV - 1,)),
            pltpu.SemaphoreType.DMA((5,)),
            pltpu.SemaphoreType.DMA((5,)),
            pltpu.SemaphoreType.DMA((2, 2)),
        ],
        compiler_params=pltpu.CompilerParams(
            collective_id=0,
            vmem_limit_bytes=100 * 1024 * 1024,
        ),
    )(x, W1, W2)


def kernel(x, W1, W2):
    return _fused(x, W1, W2)
